# baseline (speedup 1.0000x reference)
import functools

import jax
import jax.numpy as jnp
from jax import lax
from jax.experimental import pallas as pl
from jax.experimental.pallas import tpu as pltpu
from jax.experimental.pallas import tpu_sc as plsc

N = 10000
D = 128
F = 64
G = 64
OUT = 64
E = 320000

NT = 10240
ROWS_PER_SUB = NT // 16
CHUNK = 128
NW = 32
EP = E + N
T_CH = 4 * (-(-EP // (NW * CHUNK * 4)))
T_C0 = 60
T_C1 = 2 * T_CH - T_C0
EPAD = 16 * (T_C0 + T_C1) * CHUNK


def _prep1_body(x_ref, w_ref, as_ref, ad_ref, h_ref, asv_ref, adv_ref):
    x = x_ref[...]
    h = jnp.dot(x, w_ref[...], preferred_element_type=jnp.float32)
    h_ref[...] = h
    asv_ref[...] = jnp.dot(h, as_ref[...], preferred_element_type=jnp.float32)
    adv_ref[...] = jnp.dot(h, ad_ref[...], preferred_element_type=jnp.float32)


def _prep_next_body(np_ref, dp_ref, b_ref, w_ref, as_ref, ad_ref,
                    x_ref, h_ref, asv_ref, adv_ref):
    num = np_ref[0] + np_ref[1]
    den = dp_ref[0] + dp_ref[1]
    x = jnp.maximum(num / (den + 1e-16) + b_ref[...], 0.0)
    x_ref[...] = x
    h = jnp.dot(x, w_ref[...], preferred_element_type=jnp.float32)
    h_ref[...] = h
    asv_ref[...] = jnp.dot(h, as_ref[...], preferred_element_type=jnp.float32)
    adv_ref[...] = jnp.dot(h, ad_ref[...], preferred_element_type=jnp.float32)


def _final_body(np_ref, dp_ref, b_ref, x1_ref, x2_ref, batch_ref,
                wl_ref, bl_ref, out_ref):
    num = np_ref[0] + np_ref[1]
    den = dp_ref[0] + dp_ref[1]
    x3 = jnp.maximum(num / (den + 1e-16) + b_ref[...], 0.0)
    y = (x1_ref[...] + x2_ref[...] + x3) * (1.0 / 3.0)
    onehot = (batch_ref[...] == lax.broadcasted_iota(jnp.int32, (NT, G), 1)
              ).astype(jnp.float32)
    cdims = (((0,), (0,)), ((), ()))
    sums = lax.dot_general(onehot, y, cdims, preferred_element_type=jnp.float32)
    counts = lax.dot_general(onehot, jnp.ones((NT, 1), jnp.float32), cdims,
                             preferred_element_type=jnp.float32)
    pooled = sums / jnp.maximum(counts, 1.0)
    logits = jnp.dot(pooled, wl_ref[...], preferred_element_type=jnp.float32)
    logits = logits + bl_ref[...]
    m = jnp.max(logits, axis=1, keepdims=True)
    z = jnp.exp(logits - m)
    out_ref[...] = z / jnp.sum(z, axis=1, keepdims=True)


def _prep1(x, W, a_s, a_d):
    return pl.pallas_call(
        _prep1_body,
        out_shape=(jax.ShapeDtypeStruct((NT, F), jnp.float32),
                   jax.ShapeDtypeStruct((NT, 1), jnp.float32),
                   jax.ShapeDtypeStruct((NT, 1), jnp.float32)),
    )(x, W, a_s, a_d)


def _prep_next(num_p, den_p, b, W, a_s, a_d):
    return pl.pallas_call(
        _prep_next_body,
        out_shape=(jax.ShapeDtypeStruct((NT, F), jnp.float32),
                   jax.ShapeDtypeStruct((NT, F), jnp.float32),
                   jax.ShapeDtypeStruct((NT, 1), jnp.float32),
                   jax.ShapeDtypeStruct((NT, 1), jnp.float32)),
    )(num_p, den_p, b, W, a_s, a_d)


def _final(num_p, den_p, b, x1, x2, batch_col, Wl, bl):
    return pl.pallas_call(
        _final_body,
        out_shape=jax.ShapeDtypeStruct((G, OUT), jnp.float32),
    )(num_p, den_p, b, x1, x2, batch_col, Wl, bl)


_MESH = plsc.VectorSubcoreMesh(core_axis_name="c", subcore_axis_name="s")


def _edge_body(src_hbm, dst_hbm, h_hbm, asv_hbm, adv_hbm, z2_hbm, z1_hbm,
               num_out, den_out,
               asv_v, adv_v, idx_s, idx_d, wv, rows_g, rows_s,
               num_sp, den_sp, sem_i, sem_g, sem_sr, sem_sw):
    cid = lax.axis_index("c")
    sid = lax.axis_index("s")
    my_t = jnp.where(cid == 0, T_C0, T_C1)
    chunk0 = jnp.where(cid == 0, sid * T_C0, 16 * T_C0 + sid * T_C1)
    base_n = sid * ROWS_PER_SUB

    pltpu.sync_copy(z2_hbm, num_sp.at[pl.ds(base_n, ROWS_PER_SUB)])
    pltpu.sync_copy(z1_hbm, den_sp.at[pl.ds(base_n, ROWS_PER_SUB)])
    pltpu.sync_copy(asv_hbm, asv_v)
    pltpu.sync_copy(adv_hbm, adv_v)
    plsc.subcore_barrier()

    def idx_copies(t, slot):
        base = (chunk0 + t) * CHUNK
        return (pltpu.make_async_copy(src_hbm.at[pl.ds(base, CHUNK)],
                                      idx_s.at[slot], sem_i.at[slot]),
                pltpu.make_async_copy(dst_hbm.at[pl.ds(base, CHUNK)],
                                      idx_d.at[slot], sem_i.at[slot]))

    def gather_copy(slot4, b2):
        return pltpu.make_async_copy(h_hbm.at[idx_s.at[slot4]],
                                     rows_g.at[b2], sem_g.at[b2])

    def scatter_copies(slot4, b2):
        return (pltpu.make_async_copy(rows_s.at[b2],
                                      num_sp.at[idx_d.at[slot4]],
                                      sem_sr.at[b2]),
                pltpu.make_async_copy(wv.at[b2],
                                      den_sp.at[idx_d.at[slot4]],
                                      sem_sw.at[b2]))

    def compute_w(slot4, b2):
        for j in range(CHUNK // 16):
            si = idx_s[slot4, pl.ds(j * 16, 16)]
            di = idx_d[slot4, pl.ds(j * 16, 16)]
            e = plsc.load_gather(asv_v, [si]) + plsc.load_gather(adv_v, [di])
            e = jnp.maximum(e, 0.2 * e)
            wv[b2, pl.ds(j * 16, 16)] = jnp.exp(e)

    def scale(b2):
        def g_body(g, c2):
            wvec = wv[b2, pl.ds(g * 16, 16)]
            for el in range(16):
                i = g * 16 + el
                wb = jnp.full((16,), wvec[el], jnp.float32)
                for q in range(F // 16):
                    rows_s[b2, i, pl.ds(q * 16, 16)] = (
                        rows_g[b2, i, pl.ds(q * 16, 16)] * wb)
            return c2

        lax.fori_loop(0, CHUNK // 16, g_body, 0, unroll=False)

    for cp in idx_copies(0, 0):
        cp.start()
        cp.wait()
    gather_copy(0, 0).start()
    for cp in idx_copies(1, 1):
        cp.start()

    def quad_body(t4, carry):
        for b in range(4):
            t = t4 * 4 + b
            b2 = b % 2
            nb2 = 1 - b2
            s_cur = b
            s_next = (b + 1) % 4
            s_pre = (b + 2) % 4

            @pl.when(t + 1 < my_t)
            def _():
                for cp in idx_copies(t + 1, s_next):
                    cp.wait()
                gather_copy(s_next, nb2).start()

            @pl.when(t >= 2)
            def _():
                for cp in scatter_copies(s_pre, b2):
                    cp.wait()

            @pl.when(t + 2 < my_t)
            def _():
                for cp in idx_copies(t + 2, s_pre):
                    cp.start()

            compute_w(s_cur, b2)
            gather_copy(s_cur, b2).wait()
            scale(b2)
            for cp in scatter_copies(s_cur, b2):
                cp.start(add=True)
        return carry

    lax.fori_loop(0, my_t // 4, quad_body, 0, unroll=False)
    for cp in scatter_copies(2, 0):
        cp.wait()
    for cp in scatter_copies(3, 1):
        cp.wait()

    plsc.subcore_barrier()
    pltpu.sync_copy(num_sp.at[pl.ds(base_n, ROWS_PER_SUB)],
                    num_out.at[cid, pl.ds(base_n, ROWS_PER_SUB)])
    pltpu.sync_copy(den_sp.at[pl.ds(base_n, ROWS_PER_SUB)],
                    den_out.at[cid, pl.ds(base_n, ROWS_PER_SUB)])


_edge_agg = functools.partial(
    pl.kernel,
    mesh=_MESH,
    compiler_params=pltpu.CompilerParams(needs_layout_passes=False,
                                         use_tc_tiling_on_sc=False),
    out_type=(jax.ShapeDtypeStruct((2, NT, F), jnp.float32),
              jax.ShapeDtypeStruct((2, NT), jnp.float32)),
    scratch_types=[
        pltpu.VMEM((NT,), jnp.float32),
        pltpu.VMEM((NT,), jnp.float32),
        pltpu.VMEM((4, CHUNK), jnp.int32),
        pltpu.VMEM((4, CHUNK), jnp.int32),
        pltpu.VMEM((2, CHUNK), jnp.float32),
        pltpu.VMEM((2, CHUNK, F), jnp.float32),
        pltpu.VMEM((2, CHUNK, F), jnp.float32),
        pltpu.VMEM_SHARED((NT, F), jnp.float32),
        pltpu.VMEM_SHARED((NT,), jnp.float32),
        pltpu.SemaphoreType.DMA((4,)),
        pltpu.SemaphoreType.DMA((2,)),
        pltpu.SemaphoreType.DMA((2,)),
        pltpu.SemaphoreType.DMA((2,)),
    ],
)(_edge_body)


def kernel(X, L, batch, W1, a_s1, a_d1, b1, W2, a_s2, a_d2, b2,
           W3, a_s3, a_d3, b3, Wl, bl):
    ei = L[0]
    loop = jnp.arange(N, dtype=ei.dtype)
    src = jnp.concatenate([ei[0], loop])
    dst = jnp.concatenate([ei[1], loop])
    src = jnp.pad(src, (0, EPAD - EP), constant_values=N)
    dst = jnp.pad(dst, (0, EPAD - EP), constant_values=N)

    x0 = jnp.pad(X[0], ((0, NT - N), (0, 0)))
    batch_col = jnp.pad(batch[0], (0, NT - N), constant_values=G).reshape(NT, 1)
    z2 = jnp.zeros((ROWS_PER_SUB, F), jnp.float32)
    z1 = jnp.zeros((ROWS_PER_SUB,), jnp.float32)

    def layer_edges(h, asv, adv):
        return _edge_agg(src, dst, h, asv.reshape(NT), adv.reshape(NT), z2, z1)

    h1, asv1, adv1 = _prep1(x0, W1, a_s1.reshape(F, 1), a_d1.reshape(F, 1))
    np1, dn1 = layer_edges(h1, asv1, adv1)
    x1, h2, asv2, adv2 = _prep_next(np1, dn1.reshape(2, NT, 1),
                                    b1.reshape(1, F), W2,
                                    a_s2.reshape(F, 1), a_d2.reshape(F, 1))
    np2, dn2 = layer_edges(h2, asv2, adv2)
    x2, h3, asv3, adv3 = _prep_next(np2, dn2.reshape(2, NT, 1),
                                    b2.reshape(1, F), W3,
                                    a_s3.reshape(F, 1), a_d3.reshape(F, 1))
    np3, dn3 = layer_edges(h3, asv3, adv3)
    return _final(np3, dn3.reshape(2, NT, 1), b3.reshape(1, F),
                  x1, x2, batch_col, Wl, bl)


# speedup vs baseline: 1.0010x; 1.0010x over previous
import functools

import jax
import jax.numpy as jnp
from jax import lax
from jax.experimental import pallas as pl
from jax.experimental.pallas import tpu as pltpu
from jax.experimental.pallas import tpu_sc as plsc

N = 10000
D = 128
F = 64
G = 64
OUT = 64
E = 320000

NT = 10240
ROWS_PER_SUB = NT // 16
CHUNK = 128
NW = 32
EP = E + N
T_CH = 4 * (-(-EP // (NW * CHUNK * 4)))
T_C0 = 84
T_C1 = 2 * T_CH - T_C0
EPAD = 16 * (T_C0 + T_C1) * CHUNK


def _prep1_body(x_ref, w_ref, as_ref, ad_ref, h_ref, asv_ref, adv_ref):
    x = x_ref[...]
    h = jnp.dot(x, w_ref[...], preferred_element_type=jnp.float32)
    h_ref[...] = h
    asv_ref[...] = jnp.dot(h, as_ref[...], preferred_element_type=jnp.float32)
    adv_ref[...] = jnp.dot(h, ad_ref[...], preferred_element_type=jnp.float32)


def _prep_next_body(np_ref, dp_ref, b_ref, w_ref, as_ref, ad_ref,
                    x_ref, h_ref, asv_ref, adv_ref):
    num = np_ref[0] + np_ref[1]
    den = dp_ref[0] + dp_ref[1]
    x = jnp.maximum(num / (den + 1e-16) + b_ref[...], 0.0)
    x_ref[...] = x
    h = jnp.dot(x, w_ref[...], preferred_element_type=jnp.float32)
    h_ref[...] = h
    asv_ref[...] = jnp.dot(h, as_ref[...], preferred_element_type=jnp.float32)
    adv_ref[...] = jnp.dot(h, ad_ref[...], preferred_element_type=jnp.float32)


def _final_body(np_ref, dp_ref, b_ref, x1_ref, x2_ref, batch_ref,
                wl_ref, bl_ref, out_ref):
    num = np_ref[0] + np_ref[1]
    den = dp_ref[0] + dp_ref[1]
    x3 = jnp.maximum(num / (den + 1e-16) + b_ref[...], 0.0)
    y = (x1_ref[...] + x2_ref[...] + x3) * (1.0 / 3.0)
    onehot = (batch_ref[...] == lax.broadcasted_iota(jnp.int32, (NT, G), 1)
              ).astype(jnp.float32)
    cdims = (((0,), (0,)), ((), ()))
    sums = lax.dot_general(onehot, y, cdims, preferred_element_type=jnp.float32)
    counts = lax.dot_general(onehot, jnp.ones((NT, 1), jnp.float32), cdims,
                             preferred_element_type=jnp.float32)
    pooled = sums / jnp.maximum(counts, 1.0)
    logits = jnp.dot(pooled, wl_ref[...], preferred_element_type=jnp.float32)
    logits = logits + bl_ref[...]
    m = jnp.max(logits, axis=1, keepdims=True)
    z = jnp.exp(logits - m)
    out_ref[...] = z / jnp.sum(z, axis=1, keepdims=True)


def _prep1(x, W, a_s, a_d):
    return pl.pallas_call(
        _prep1_body,
        out_shape=(jax.ShapeDtypeStruct((NT, F), jnp.float32),
                   jax.ShapeDtypeStruct((NT, 1), jnp.float32),
                   jax.ShapeDtypeStruct((NT, 1), jnp.float32)),
    )(x, W, a_s, a_d)


def _prep_next(num_p, den_p, b, W, a_s, a_d):
    return pl.pallas_call(
        _prep_next_body,
        out_shape=(jax.ShapeDtypeStruct((NT, F), jnp.float32),
                   jax.ShapeDtypeStruct((NT, F), jnp.float32),
                   jax.ShapeDtypeStruct((NT, 1), jnp.float32),
                   jax.ShapeDtypeStruct((NT, 1), jnp.float32)),
    )(num_p, den_p, b, W, a_s, a_d)


def _final(num_p, den_p, b, x1, x2, batch_col, Wl, bl):
    return pl.pallas_call(
        _final_body,
        out_shape=jax.ShapeDtypeStruct((G, OUT), jnp.float32),
    )(num_p, den_p, b, x1, x2, batch_col, Wl, bl)


_MESH = plsc.VectorSubcoreMesh(core_axis_name="c", subcore_axis_name="s")


def _edge_body(src_hbm, dst_hbm, h_hbm, asv_hbm, adv_hbm, z2_hbm, z1_hbm,
               num_out, den_out,
               asv_v, adv_v, idx_s, idx_d, wv, rows_g, rows_s,
               num_sp, den_sp, sem_i, sem_g, sem_sr, sem_sw):
    cid = lax.axis_index("c")
    sid = lax.axis_index("s")
    my_t = jnp.where(cid == 0, T_C0, T_C1)
    chunk0 = jnp.where(cid == 0, sid * T_C0, 16 * T_C0 + sid * T_C1)
    base_n = sid * ROWS_PER_SUB

    pltpu.sync_copy(z2_hbm, num_sp.at[pl.ds(base_n, ROWS_PER_SUB)])
    pltpu.sync_copy(z1_hbm, den_sp.at[pl.ds(base_n, ROWS_PER_SUB)])
    pltpu.sync_copy(asv_hbm, asv_v)
    pltpu.sync_copy(adv_hbm, adv_v)
    plsc.subcore_barrier()

    def idx_copies(t, slot):
        base = (chunk0 + t) * CHUNK
        return (pltpu.make_async_copy(src_hbm.at[pl.ds(base, CHUNK)],
                                      idx_s.at[slot], sem_i.at[slot]),
                pltpu.make_async_copy(dst_hbm.at[pl.ds(base, CHUNK)],
                                      idx_d.at[slot], sem_i.at[slot]))

    def gather_copy(slot4, b2):
        return pltpu.make_async_copy(h_hbm.at[idx_s.at[slot4]],
                                     rows_g.at[b2], sem_g.at[b2])

    def scatter_copies(slot4, b2):
        return (pltpu.make_async_copy(rows_s.at[b2],
                                      num_sp.at[idx_d.at[slot4]],
                                      sem_sr.at[b2]),
                pltpu.make_async_copy(wv.at[b2],
                                      den_sp.at[idx_d.at[slot4]],
                                      sem_sw.at[b2]))

    def compute_w(slot4, b2):
        for j in range(CHUNK // 16):
            si = idx_s[slot4, pl.ds(j * 16, 16)]
            di = idx_d[slot4, pl.ds(j * 16, 16)]
            e = plsc.load_gather(asv_v, [si]) + plsc.load_gather(adv_v, [di])
            e = jnp.maximum(e, 0.2 * e)
            wv[b2, pl.ds(j * 16, 16)] = jnp.exp(e)

    def scale(b2):
        def g_body(g, c2):
            wvec = wv[b2, pl.ds(g * 16, 16)]
            for el in range(16):
                i = g * 16 + el
                wb = jnp.full((16,), wvec[el], jnp.float32)
                for q in range(F // 16):
                    rows_s[b2, i, pl.ds(q * 16, 16)] = (
                        rows_g[b2, i, pl.ds(q * 16, 16)] * wb)
            return c2

        lax.fori_loop(0, CHUNK // 16, g_body, 0, unroll=False)

    for cp in idx_copies(0, 0):
        cp.start()
        cp.wait()
    gather_copy(0, 0).start()
    for cp in idx_copies(1, 1):
        cp.start()

    def quad_body(t4, carry):
        for b in range(4):
            t = t4 * 4 + b
            b2 = b % 2
            nb2 = 1 - b2
            s_cur = b
            s_next = (b + 1) % 4
            s_pre = (b + 2) % 4

            @pl.when(t + 1 < my_t)
            def _():
                for cp in idx_copies(t + 1, s_next):
                    cp.wait()
                gather_copy(s_next, nb2).start()

            @pl.when(t >= 2)
            def _():
                for cp in scatter_copies(s_pre, b2):
                    cp.wait()

            @pl.when(t + 2 < my_t)
            def _():
                for cp in idx_copies(t + 2, s_pre):
                    cp.start()

            compute_w(s_cur, b2)
            gather_copy(s_cur, b2).wait()
            scale(b2)
            for cp in scatter_copies(s_cur, b2):
                cp.start(add=True)
        return carry

    lax.fori_loop(0, my_t // 4, quad_body, 0, unroll=False)
    for cp in scatter_copies(2, 0):
        cp.wait()
    for cp in scatter_copies(3, 1):
        cp.wait()

    plsc.subcore_barrier()
    pltpu.sync_copy(num_sp.at[pl.ds(base_n, ROWS_PER_SUB)],
                    num_out.at[cid, pl.ds(base_n, ROWS_PER_SUB)])
    pltpu.sync_copy(den_sp.at[pl.ds(base_n, ROWS_PER_SUB)],
                    den_out.at[cid, pl.ds(base_n, ROWS_PER_SUB)])


_edge_agg = functools.partial(
    pl.kernel,
    mesh=_MESH,
    compiler_params=pltpu.CompilerParams(needs_layout_passes=False,
                                         use_tc_tiling_on_sc=False),
    out_type=(jax.ShapeDtypeStruct((2, NT, F), jnp.float32),
              jax.ShapeDtypeStruct((2, NT), jnp.float32)),
    scratch_types=[
        pltpu.VMEM((NT,), jnp.float32),
        pltpu.VMEM((NT,), jnp.float32),
        pltpu.VMEM((4, CHUNK), jnp.int32),
        pltpu.VMEM((4, CHUNK), jnp.int32),
        pltpu.VMEM((2, CHUNK), jnp.float32),
        pltpu.VMEM((2, CHUNK, F), jnp.float32),
        pltpu.VMEM((2, CHUNK, F), jnp.float32),
        pltpu.VMEM_SHARED((NT, F), jnp.float32),
        pltpu.VMEM_SHARED((NT,), jnp.float32),
        pltpu.SemaphoreType.DMA((4,)),
        pltpu.SemaphoreType.DMA((2,)),
        pltpu.SemaphoreType.DMA((2,)),
        pltpu.SemaphoreType.DMA((2,)),
    ],
)(_edge_body)


def kernel(X, L, batch, W1, a_s1, a_d1, b1, W2, a_s2, a_d2, b2,
           W3, a_s3, a_d3, b3, Wl, bl):
    ei = L[0]
    loop = jnp.arange(N, dtype=ei.dtype)
    src = jnp.concatenate([ei[0], loop])
    dst = jnp.concatenate([ei[1], loop])
    src = jnp.pad(src, (0, EPAD - EP), constant_values=N)
    dst = jnp.pad(dst, (0, EPAD - EP), constant_values=N)

    x0 = jnp.pad(X[0], ((0, NT - N), (0, 0)))
    batch_col = jnp.pad(batch[0], (0, NT - N), constant_values=G).reshape(NT, 1)
    z2 = jnp.zeros((ROWS_PER_SUB, F), jnp.float32)
    z1 = jnp.zeros((ROWS_PER_SUB,), jnp.float32)

    def layer_edges(h, asv, adv):
        return _edge_agg(src, dst, h, asv.reshape(NT), adv.reshape(NT), z2, z1)

    h1, asv1, adv1 = _prep1(x0, W1, a_s1.reshape(F, 1), a_d1.reshape(F, 1))
    np1, dn1 = layer_edges(h1, asv1, adv1)
    x1, h2, asv2, adv2 = _prep_next(np1, dn1.reshape(2, NT, 1),
                                    b1.reshape(1, F), W2,
                                    a_s2.reshape(F, 1), a_d2.reshape(F, 1))
    np2, dn2 = layer_edges(h2, asv2, adv2)
    x2, h3, asv3, adv3 = _prep_next(np2, dn2.reshape(2, NT, 1),
                                    b2.reshape(1, F), W3,
                                    a_s3.reshape(F, 1), a_d3.reshape(F, 1))
    np3, dn3 = layer_edges(h3, asv3, adv3)
    return _final(np3, dn3.reshape(2, NT, 1), b3.reshape(1, F),
                  x1, x2, batch_col, Wl, bl)


# speedup vs baseline: 2.5396x; 2.5371x over previous
import functools

import jax
import jax.numpy as jnp
from jax import lax
from jax.experimental import pallas as pl
from jax.experimental.pallas import tpu as pltpu
from jax.experimental.pallas import tpu_sc as plsc

N = 10000
D = 128
F = 64
G = 64
OUT = 64
E = 320000

NT = 10240
ROWS_PER_SUB = NT // 16
CHUNK = 128
FH = F // 2
EP = E + N
T_CH = 4 * (-(-EP // (16 * CHUNK * 4)))
EPAD = 16 * T_CH * CHUNK


def _prep1_body(x_ref, w_ref, as_ref, ad_ref, h_ref, asv_ref, adv_ref):
    x = x_ref[...]
    h = jnp.dot(x, w_ref[...], preferred_element_type=jnp.float32)
    h_ref[0] = h[:, :FH]
    h_ref[1] = h[:, FH:]
    asv_ref[...] = jnp.dot(h, as_ref[...], preferred_element_type=jnp.float32)
    adv_ref[...] = jnp.dot(h, ad_ref[...], preferred_element_type=jnp.float32)


def _prep_next_body(np_ref, dp_ref, b_ref, w_ref, as_ref, ad_ref,
                    x_ref, h_ref, asv_ref, adv_ref):
    num = jnp.concatenate([np_ref[0], np_ref[1]], axis=-1)
    den = dp_ref[0]
    x = jnp.maximum(num / (den + 1e-16) + b_ref[...], 0.0)
    x_ref[...] = x
    h = jnp.dot(x, w_ref[...], preferred_element_type=jnp.float32)
    h_ref[0] = h[:, :FH]
    h_ref[1] = h[:, FH:]
    asv_ref[...] = jnp.dot(h, as_ref[...], preferred_element_type=jnp.float32)
    adv_ref[...] = jnp.dot(h, ad_ref[...], preferred_element_type=jnp.float32)


def _final_body(np_ref, dp_ref, b_ref, x1_ref, x2_ref, batch_ref,
                wl_ref, bl_ref, out_ref):
    num = jnp.concatenate([np_ref[0], np_ref[1]], axis=-1)
    den = dp_ref[0]
    x3 = jnp.maximum(num / (den + 1e-16) + b_ref[...], 0.0)
    y = (x1_ref[...] + x2_ref[...] + x3) * (1.0 / 3.0)
    onehot = (batch_ref[...] == lax.broadcasted_iota(jnp.int32, (NT, G), 1)
              ).astype(jnp.float32)
    cdims = (((0,), (0,)), ((), ()))
    sums = lax.dot_general(onehot, y, cdims, preferred_element_type=jnp.float32)
    counts = lax.dot_general(onehot, jnp.ones((NT, 1), jnp.float32), cdims,
                             preferred_element_type=jnp.float32)
    pooled = sums / jnp.maximum(counts, 1.0)
    logits = jnp.dot(pooled, wl_ref[...], preferred_element_type=jnp.float32)
    logits = logits + bl_ref[...]
    m = jnp.max(logits, axis=1, keepdims=True)
    z = jnp.exp(logits - m)
    out_ref[...] = z / jnp.sum(z, axis=1, keepdims=True)


def _prep1(x, W, a_s, a_d):
    return pl.pallas_call(
        _prep1_body,
        out_shape=(jax.ShapeDtypeStruct((2, NT, FH), jnp.float32),
                   jax.ShapeDtypeStruct((NT, 1), jnp.float32),
                   jax.ShapeDtypeStruct((NT, 1), jnp.float32)),
    )(x, W, a_s, a_d)


def _prep_next(num_p, den_p, b, W, a_s, a_d):
    return pl.pallas_call(
        _prep_next_body,
        out_shape=(jax.ShapeDtypeStruct((NT, F), jnp.float32),
                   jax.ShapeDtypeStruct((2, NT, FH), jnp.float32),
                   jax.ShapeDtypeStruct((NT, 1), jnp.float32),
                   jax.ShapeDtypeStruct((NT, 1), jnp.float32)),
    )(num_p, den_p, b, W, a_s, a_d)


def _final(num_p, den_p, b, x1, x2, batch_col, Wl, bl):
    return pl.pallas_call(
        _final_body,
        out_shape=jax.ShapeDtypeStruct((G, OUT), jnp.float32),
    )(num_p, den_p, b, x1, x2, batch_col, Wl, bl)


_MESH = plsc.VectorSubcoreMesh(core_axis_name="c", subcore_axis_name="s")


def _edge_body(src_hbm, dst_hbm, h_hbm, asv_hbm, adv_hbm, z2_hbm, z1_hbm,
               num_out, den_out,
               asv_v, adv_v, idx_s, idx_d, wv, rows_g, rows_s,
               h_sp, num_sp, den_sp, sem_i, sem_g, sem_sr, sem_sw):
    cid = lax.axis_index("c")
    sid = lax.axis_index("s")
    base_n = sid * ROWS_PER_SUB

    pltpu.sync_copy(z2_hbm, num_sp.at[pl.ds(base_n, ROWS_PER_SUB)])
    pltpu.sync_copy(z1_hbm, den_sp.at[pl.ds(base_n, ROWS_PER_SUB)])
    pltpu.sync_copy(h_hbm.at[cid, pl.ds(base_n, ROWS_PER_SUB)],
                    h_sp.at[pl.ds(base_n, ROWS_PER_SUB)])
    pltpu.sync_copy(asv_hbm, asv_v)
    pltpu.sync_copy(adv_hbm, adv_v)
    plsc.subcore_barrier()

    def idx_copies(t, slot):
        base = (sid * T_CH + t) * CHUNK
        return (pltpu.make_async_copy(src_hbm.at[pl.ds(base, CHUNK)],
                                      idx_s.at[slot], sem_i.at[slot]),
                pltpu.make_async_copy(dst_hbm.at[pl.ds(base, CHUNK)],
                                      idx_d.at[slot], sem_i.at[slot]))

    def gather_copy(slot4, b2):
        return pltpu.make_async_copy(h_sp.at[idx_s.at[slot4]],
                                     rows_g.at[b2], sem_g.at[b2])

    def scatter_copies(slot4, b2):
        return (pltpu.make_async_copy(rows_s.at[b2],
                                      num_sp.at[idx_d.at[slot4]],
                                      sem_sr.at[b2]),
                pltpu.make_async_copy(wv.at[b2],
                                      den_sp.at[idx_d.at[slot4]],
                                      sem_sw.at[b2]))

    def compute_w(slot4, b2):
        for j in range(CHUNK // 16):
            si = idx_s[slot4, pl.ds(j * 16, 16)]
            di = idx_d[slot4, pl.ds(j * 16, 16)]
            e = plsc.load_gather(asv_v, [si]) + plsc.load_gather(adv_v, [di])
            e = jnp.maximum(e, 0.2 * e)
            wv[b2, pl.ds(j * 16, 16)] = jnp.exp(e)

    def scale(b2):
        def g_body(g, c2):
            wvec = wv[b2, pl.ds(g * 16, 16)]
            for el in range(16):
                i = g * 16 + el
                wb = jnp.full((16,), wvec[el], jnp.float32)
                for q in range(FH // 16):
                    rows_s[b2, i, pl.ds(q * 16, 16)] = (
                        rows_g[b2, i, pl.ds(q * 16, 16)] * wb)
            return c2

        lax.fori_loop(0, CHUNK // 16, g_body, 0, unroll=False)

    for cp in idx_copies(0, 0):
        cp.start()
        cp.wait()
    gather_copy(0, 0).start()
    for cp in idx_copies(1, 1):
        cp.start()

    def quad_body(t4, carry):
        for b in range(4):
            t = t4 * 4 + b
            b2 = b % 2
            nb2 = 1 - b2
            s_cur = b
            s_next = (b + 1) % 4
            s_pre = (b + 2) % 4

            @pl.when(t + 1 < T_CH)
            def _():
                for cp in idx_copies(t + 1, s_next):
                    cp.wait()
                gather_copy(s_next, nb2).start()

            @pl.when(t >= 2)
            def _():
                for cp in scatter_copies(s_pre, b2):
                    cp.wait()

            @pl.when(t + 2 < T_CH)
            def _():
                for cp in idx_copies(t + 2, s_pre):
                    cp.start()

            compute_w(s_cur, b2)
            gather_copy(s_cur, b2).wait()
            scale(b2)
            for cp in scatter_copies(s_cur, b2):
                cp.start(add=True)
        return carry

    lax.fori_loop(0, T_CH // 4, quad_body, 0, unroll=False)
    for cp in scatter_copies(2, 0):
        cp.wait()
    for cp in scatter_copies(3, 1):
        cp.wait()

    plsc.subcore_barrier()
    pltpu.sync_copy(num_sp.at[pl.ds(base_n, ROWS_PER_SUB)],
                    num_out.at[cid, pl.ds(base_n, ROWS_PER_SUB)])
    pltpu.sync_copy(den_sp.at[pl.ds(base_n, ROWS_PER_SUB)],
                    den_out.at[cid, pl.ds(base_n, ROWS_PER_SUB)])


_edge_agg = functools.partial(
    pl.kernel,
    mesh=_MESH,
    compiler_params=pltpu.CompilerParams(needs_layout_passes=False,
                                         use_tc_tiling_on_sc=False),
    out_type=(jax.ShapeDtypeStruct((2, NT, FH), jnp.float32),
              jax.ShapeDtypeStruct((2, NT), jnp.float32)),
    scratch_types=[
        pltpu.VMEM((NT,), jnp.float32),
        pltpu.VMEM((NT,), jnp.float32),
        pltpu.VMEM((4, CHUNK), jnp.int32),
        pltpu.VMEM((4, CHUNK), jnp.int32),
        pltpu.VMEM((2, CHUNK), jnp.float32),
        pltpu.VMEM((2, CHUNK, FH), jnp.float32),
        pltpu.VMEM((2, CHUNK, FH), jnp.float32),
        pltpu.VMEM_SHARED((NT, FH), jnp.float32),
        pltpu.VMEM_SHARED((NT, FH), jnp.float32),
        pltpu.VMEM_SHARED((NT,), jnp.float32),
        pltpu.SemaphoreType.DMA((4,)),
        pltpu.SemaphoreType.DMA((2,)),
        pltpu.SemaphoreType.DMA((2,)),
        pltpu.SemaphoreType.DMA((2,)),
    ],
)(_edge_body)


def kernel(X, L, batch, W1, a_s1, a_d1, b1, W2, a_s2, a_d2, b2,
           W3, a_s3, a_d3, b3, Wl, bl):
    ei = L[0]
    loop = jnp.arange(N, dtype=ei.dtype)
    src = jnp.concatenate([ei[0], loop])
    dst = jnp.concatenate([ei[1], loop])
    src = jnp.pad(src, (0, EPAD - EP), constant_values=N)
    dst = jnp.pad(dst, (0, EPAD - EP), constant_values=N)

    x0 = jnp.pad(X[0], ((0, NT - N), (0, 0)))
    batch_col = jnp.pad(batch[0], (0, NT - N), constant_values=G).reshape(NT, 1)
    z2 = jnp.zeros((ROWS_PER_SUB, FH), jnp.float32)
    z1 = jnp.zeros((ROWS_PER_SUB,), jnp.float32)

    def layer_edges(h, asv, adv):
        return _edge_agg(src, dst, h, asv.reshape(NT), adv.reshape(NT), z2, z1)

    h1, asv1, adv1 = _prep1(x0, W1, a_s1.reshape(F, 1), a_d1.reshape(F, 1))
    np1, dn1 = layer_edges(h1, asv1, adv1)
    x1, h2, asv2, adv2 = _prep_next(np1, dn1.reshape(2, NT, 1),
                                    b1.reshape(1, F), W2,
                                    a_s2.reshape(F, 1), a_d2.reshape(F, 1))
    np2, dn2 = layer_edges(h2, asv2, adv2)
    x2, h3, asv3, adv3 = _prep_next(np2, dn2.reshape(2, NT, 1),
                                    b2.reshape(1, F), W3,
                                    a_s3.reshape(F, 1), a_d3.reshape(F, 1))
    np3, dn3 = layer_edges(h3, asv3, adv3)
    return _final(np3, dn3.reshape(2, NT, 1), b3.reshape(1, F),
                  x1, x2, batch_col, Wl, bl)


# speedup vs baseline: 2.6233x; 1.0329x over previous
import functools

import jax
import jax.numpy as jnp
from jax import lax
from jax.experimental import pallas as pl
from jax.experimental.pallas import tpu as pltpu
from jax.experimental.pallas import tpu_sc as plsc

N = 10000
D = 128
F = 64
G = 64
OUT = 64
E = 320000

NT = 10240
ROWS_PER_SUB = NT // 16
CHUNK = 128
FH = F // 2
T_CH = 4 * (-(-E // (16 * CHUNK * 4)))
EPAD = 16 * T_CH * CHUNK


def _h_proj(x, w_ref, as_ref, ad_ref, h_ref, asv_ref, adv_ref):
    h = jnp.dot(x, w_ref[...], preferred_element_type=jnp.float32)
    h_ref[0] = h[:, :FH]
    h_ref[1] = h[:, FH:]
    asv_ref[...] = jnp.dot(h, as_ref[...],
                           preferred_element_type=jnp.float32)[:, 0]
    adv_ref[...] = jnp.dot(h, ad_ref[...],
                           preferred_element_type=jnp.float32)[:, 0]


def _gat_epilogue(np_ref, dp_ref, hp_ref, asv_ref, adv_ref, b_ref):
    s = asv_ref[...] + adv_ref[...]
    ws = jnp.exp(jnp.maximum(s, 0.2 * s))[:, None]
    hp = jnp.concatenate([hp_ref[0], hp_ref[1]], axis=-1)
    num = jnp.concatenate([np_ref[0], np_ref[1]], axis=-1) + ws * hp
    den = dp_ref[0][:, None] + ws
    return jnp.maximum(num / (den + 1e-16) + b_ref[...], 0.0)


def _prep1_body(x_ref, w_ref, as_ref, ad_ref, h_ref, asv_ref, adv_ref):
    _h_proj(x_ref[...], w_ref, as_ref, ad_ref, h_ref, asv_ref, adv_ref)


def _prep_next_body(np_ref, dp_ref, hp_ref, pasv_ref, padv_ref, b_ref,
                    w_ref, as_ref, ad_ref,
                    x_ref, h_ref, asv_ref, adv_ref):
    x = _gat_epilogue(np_ref, dp_ref, hp_ref, pasv_ref, padv_ref, b_ref)
    x_ref[...] = x
    _h_proj(x, w_ref, as_ref, ad_ref, h_ref, asv_ref, adv_ref)


def _pool_body(np_ref, dp_ref, hp_ref, pasv_ref, padv_ref, b_ref,
               x1_ref, x2_ref, batch_ref, sums_ref, counts_ref):
    x3 = _gat_epilogue(np_ref, dp_ref, hp_ref, pasv_ref, padv_ref, b_ref)
    y = (x1_ref[...] + x2_ref[...] + x3) * (1.0 / 3.0)
    blk = batch_ref.shape[0]
    onehot = (batch_ref[...] == lax.broadcasted_iota(jnp.int32, (blk, G), 1)
              ).astype(jnp.float32)
    cdims = (((0,), (0,)), ((), ()))
    sums = lax.dot_general(onehot, y, cdims, preferred_element_type=jnp.float32)
    counts = lax.dot_general(onehot, jnp.ones((blk, 1), jnp.float32), cdims,
                             preferred_element_type=jnp.float32)

    @pl.when(pl.program_id(0) == 0)
    def _():
        sums_ref[...] = jnp.zeros_like(sums_ref)
        counts_ref[...] = jnp.zeros_like(counts_ref)

    sums_ref[...] += sums
    counts_ref[...] += counts


def _head_body(sums_ref, counts_ref, wl_ref, bl_ref, out_ref):
    pooled = sums_ref[...] / jnp.maximum(counts_ref[...], 1.0)
    logits = jnp.dot(pooled, wl_ref[...], preferred_element_type=jnp.float32)
    logits = logits + bl_ref[...]
    m = jnp.max(logits, axis=1, keepdims=True)
    z = jnp.exp(logits - m)
    out_ref[...] = z / jnp.sum(z, axis=1, keepdims=True)


def _prep1(x, W, a_s, a_d):
    return pl.pallas_call(
        _prep1_body,
        out_shape=(jax.ShapeDtypeStruct((2, NT, FH), jnp.float32),
                   jax.ShapeDtypeStruct((NT,), jnp.float32),
                   jax.ShapeDtypeStruct((NT,), jnp.float32)),
    )(x, W, a_s, a_d)


PB = 2048


def _row_specs():
    return [
        pl.BlockSpec((2, PB, FH), lambda i: (0, i, 0)),
        pl.BlockSpec((2, PB), lambda i: (0, i)),
        pl.BlockSpec((2, PB, FH), lambda i: (0, i, 0)),
        pl.BlockSpec((PB,), lambda i: (i,)),
        pl.BlockSpec((PB,), lambda i: (i,)),
        pl.BlockSpec((1, F), lambda i: (0, 0)),
    ]


def _prep_next(num_p, den_p, h_p, pasv, padv, b, W, a_s, a_d):
    return pl.pallas_call(
        _prep_next_body,
        grid=(NT // PB,),
        in_specs=_row_specs() + [
            pl.BlockSpec((F, F), lambda i: (0, 0)),
            pl.BlockSpec((F, 1), lambda i: (0, 0)),
            pl.BlockSpec((F, 1), lambda i: (0, 0)),
        ],
        out_specs=[
            pl.BlockSpec((PB, F), lambda i: (i, 0)),
            pl.BlockSpec((2, PB, FH), lambda i: (0, i, 0)),
            pl.BlockSpec((PB,), lambda i: (i,)),
            pl.BlockSpec((PB,), lambda i: (i,)),
        ],
        out_shape=(jax.ShapeDtypeStruct((NT, F), jnp.float32),
                   jax.ShapeDtypeStruct((2, NT, FH), jnp.float32),
                   jax.ShapeDtypeStruct((NT,), jnp.float32),
                   jax.ShapeDtypeStruct((NT,), jnp.float32)),
    )(num_p, den_p, h_p, pasv, padv, b, W, a_s, a_d)


def _final(num_p, den_p, h_p, pasv, padv, b, x1, x2, batch_col, Wl, bl):
    sums, counts = pl.pallas_call(
        _pool_body,
        grid=(NT // PB,),
        in_specs=_row_specs() + [
            pl.BlockSpec((PB, F), lambda i: (i, 0)),
            pl.BlockSpec((PB, F), lambda i: (i, 0)),
            pl.BlockSpec((PB, 1), lambda i: (i, 0)),
        ],
        out_specs=[
            pl.BlockSpec((G, F), lambda i: (0, 0)),
            pl.BlockSpec((G, 1), lambda i: (0, 0)),
        ],
        out_shape=(jax.ShapeDtypeStruct((G, F), jnp.float32),
                   jax.ShapeDtypeStruct((G, 1), jnp.float32)),
    )(num_p, den_p, h_p, pasv, padv, b, x1, x2, batch_col)
    return pl.pallas_call(
        _head_body,
        out_shape=jax.ShapeDtypeStruct((G, OUT), jnp.float32),
    )(sums, counts, Wl, bl)


_MESH = plsc.VectorSubcoreMesh(core_axis_name="c", subcore_axis_name="s")


def _edge_body(src_hbm, dst_hbm, h_hbm, asv_hbm, adv_hbm, z2_hbm, z1_hbm,
               num_out, den_out,
               asv_v, adv_v, idx_s, idx_d, wv, rows_g, rows_s,
               h_sp, num_sp, den_sp, sem_i, sem_g, sem_sr, sem_sw):
    cid = lax.axis_index("c")
    sid = lax.axis_index("s")
    base_n = sid * ROWS_PER_SUB

    pltpu.sync_copy(z2_hbm, num_sp.at[pl.ds(base_n, ROWS_PER_SUB)])
    pltpu.sync_copy(z1_hbm, den_sp.at[pl.ds(base_n, ROWS_PER_SUB)])
    pltpu.sync_copy(h_hbm.at[cid, pl.ds(base_n, ROWS_PER_SUB)],
                    h_sp.at[pl.ds(base_n, ROWS_PER_SUB)])
    pltpu.sync_copy(asv_hbm, asv_v)
    pltpu.sync_copy(adv_hbm, adv_v)
    plsc.subcore_barrier()

    def idx_copies(t, slot):
        base = (sid * T_CH + t) * CHUNK
        return (pltpu.make_async_copy(src_hbm.at[pl.ds(base, CHUNK)],
                                      idx_s.at[slot], sem_i.at[slot]),
                pltpu.make_async_copy(dst_hbm.at[pl.ds(base, CHUNK)],
                                      idx_d.at[slot], sem_i.at[slot]))

    def gather_copy(slot4, b2):
        return pltpu.make_async_copy(h_sp.at[idx_s.at[slot4]],
                                     rows_g.at[b2], sem_g.at[b2])

    def scatter_copies(slot4, b2):
        return (pltpu.make_async_copy(rows_s.at[b2],
                                      num_sp.at[idx_d.at[slot4]],
                                      sem_sr.at[b2]),
                pltpu.make_async_copy(wv.at[b2],
                                      den_sp.at[idx_d.at[slot4]],
                                      sem_sw.at[b2]))

    def compute_w(slot4, b2):
        for j in range(CHUNK // 16):
            si = idx_s[slot4, pl.ds(j * 16, 16)]
            di = idx_d[slot4, pl.ds(j * 16, 16)]
            e = plsc.load_gather(asv_v, [si]) + plsc.load_gather(adv_v, [di])
            e = jnp.maximum(e, 0.2 * e)
            wv[b2, pl.ds(j * 16, 16)] = jnp.exp(e)

    def scale(b2):
        for g in range(CHUNK // 16):
            wvec = wv[b2, pl.ds(g * 16, 16)]
            for el in range(16):
                i = g * 16 + el
                wb = jnp.full((16,), wvec[el], jnp.float32)
                for q in range(FH // 16):
                    rows_s[b2, i, pl.ds(q * 16, 16)] = (
                        rows_g[b2, i, pl.ds(q * 16, 16)] * wb)

    for cp in idx_copies(0, 0):
        cp.start()
        cp.wait()
    gather_copy(0, 0).start()
    for cp in idx_copies(1, 1):
        cp.start()

    def quad_body(t4, carry):
        for b in range(4):
            t = t4 * 4 + b
            b2 = b % 2
            nb2 = 1 - b2
            s_cur = b
            s_next = (b + 1) % 4
            s_pre = (b + 2) % 4

            @pl.when(t + 1 < T_CH)
            def _():
                for cp in idx_copies(t + 1, s_next):
                    cp.wait()
                gather_copy(s_next, nb2).start()

            @pl.when(t >= 2)
            def _():
                for cp in scatter_copies(s_pre, b2):
                    cp.wait()

            @pl.when(t + 2 < T_CH)
            def _():
                for cp in idx_copies(t + 2, s_pre):
                    cp.start()

            compute_w(s_cur, b2)
            gather_copy(s_cur, b2).wait()
            scale(b2)
            for cp in scatter_copies(s_cur, b2):
                cp.start(add=True)
        return carry

    lax.fori_loop(0, T_CH // 4, quad_body, 0, unroll=False)
    for cp in scatter_copies(2, 0):
        cp.wait()
    for cp in scatter_copies(3, 1):
        cp.wait()

    plsc.subcore_barrier()
    pltpu.sync_copy(num_sp.at[pl.ds(base_n, ROWS_PER_SUB)],
                    num_out.at[cid, pl.ds(base_n, ROWS_PER_SUB)])
    pltpu.sync_copy(den_sp.at[pl.ds(base_n, ROWS_PER_SUB)],
                    den_out.at[cid, pl.ds(base_n, ROWS_PER_SUB)])


_edge_agg = functools.partial(
    pl.kernel,
    mesh=_MESH,
    compiler_params=pltpu.CompilerParams(needs_layout_passes=False,
                                         use_tc_tiling_on_sc=False),
    out_type=(jax.ShapeDtypeStruct((2, NT, FH), jnp.float32),
              jax.ShapeDtypeStruct((2, NT), jnp.float32)),
    scratch_types=[
        pltpu.VMEM((NT,), jnp.float32),
        pltpu.VMEM((NT,), jnp.float32),
        pltpu.VMEM((4, CHUNK), jnp.int32),
        pltpu.VMEM((4, CHUNK), jnp.int32),
        pltpu.VMEM((2, CHUNK), jnp.float32),
        pltpu.VMEM((2, CHUNK, FH), jnp.float32),
        pltpu.VMEM((2, CHUNK, FH), jnp.float32),
        pltpu.VMEM_SHARED((NT, FH), jnp.float32),
        pltpu.VMEM_SHARED((NT, FH), jnp.float32),
        pltpu.VMEM_SHARED((NT,), jnp.float32),
        pltpu.SemaphoreType.DMA((4,)),
        pltpu.SemaphoreType.DMA((2,)),
        pltpu.SemaphoreType.DMA((2,)),
        pltpu.SemaphoreType.DMA((2,)),
    ],
)(_edge_body)


def kernel(X, L, batch, W1, a_s1, a_d1, b1, W2, a_s2, a_d2, b2,
           W3, a_s3, a_d3, b3, Wl, bl):
    ei = L[0]
    src = jnp.pad(ei[0], (0, EPAD - E), constant_values=N)
    dst = jnp.pad(ei[1], (0, EPAD - E), constant_values=N)

    x0 = jnp.pad(X[0], ((0, NT - N), (0, 0)))
    batch_col = jnp.pad(batch[0], (0, NT - N), constant_values=G).reshape(NT, 1)
    z2 = jnp.zeros((ROWS_PER_SUB, FH), jnp.float32)
    z1 = jnp.zeros((ROWS_PER_SUB,), jnp.float32)

    def layer_edges(h, asv, adv):
        return _edge_agg(src, dst, h, asv, adv, z2, z1)

    h1, asv1, adv1 = _prep1(x0, W1, a_s1.reshape(F, 1), a_d1.reshape(F, 1))
    np1, dn1 = layer_edges(h1, asv1, adv1)
    x1, h2, asv2, adv2 = _prep_next(np1, dn1, h1, asv1, adv1,
                                    b1.reshape(1, F), W2,
                                    a_s2.reshape(F, 1), a_d2.reshape(F, 1))
    np2, dn2 = layer_edges(h2, asv2, adv2)
    x2, h3, asv3, adv3 = _prep_next(np2, dn2, h2, asv2, adv2,
                                    b2.reshape(1, F), W3,
                                    a_s3.reshape(F, 1), a_d3.reshape(F, 1))
    np3, dn3 = layer_edges(h3, asv3, adv3)
    return _final(np3, dn3, h3, asv3, adv3, b3.reshape(1, F),
                  x1, x2, batch_col, Wl, bl)


# speedup vs baseline: 2.7751x; 1.0579x over previous
import functools

import jax
import jax.numpy as jnp
from jax import lax
from jax.experimental import pallas as pl
from jax.experimental.pallas import tpu as pltpu
from jax.experimental.pallas import tpu_sc as plsc

N = 10000
D = 128
F = 64
G = 64
OUT = 64
E = 320000

NT = 10240
ROWS_PER_SUB = NT // 16
CHUNK = 128
FH = F // 2
T_CH = 4 * (-(-E // (16 * CHUNK * 4)))
EPAD = 16 * T_CH * CHUNK


def _h_proj(x, w_ref, as_ref, ad_ref, h_ref, asv_ref, adv_ref):
    h = jnp.dot(x, w_ref[...], preferred_element_type=jnp.float32)
    h_ref[0] = h[:, :FH]
    h_ref[1] = h[:, FH:]
    asv_ref[...] = jnp.dot(h, as_ref[...],
                           preferred_element_type=jnp.float32)[:, 0]
    adv_ref[...] = jnp.dot(h, ad_ref[...],
                           preferred_element_type=jnp.float32)[:, 0]


def _gat_epilogue(np_ref, dp_ref, hp_ref, asv_ref, adv_ref, b_ref):
    s = asv_ref[...] + adv_ref[...]
    ws = jnp.exp(jnp.maximum(s, 0.2 * s))[:, None]
    hp = jnp.concatenate([hp_ref[0], hp_ref[1]], axis=-1)
    num = jnp.concatenate([np_ref[0], np_ref[1]], axis=-1) + ws * hp
    den = dp_ref[0][:, None] + ws
    return jnp.maximum(num / (den + 1e-16) + b_ref[...], 0.0)


def _prep1_body(x_ref, w_ref, as_ref, ad_ref, h_ref, asv_ref, adv_ref):
    _h_proj(x_ref[...], w_ref, as_ref, ad_ref, h_ref, asv_ref, adv_ref)


def _prep_next_body(np_ref, dp_ref, hp_ref, pasv_ref, padv_ref, b_ref,
                    w_ref, as_ref, ad_ref,
                    x_ref, h_ref, asv_ref, adv_ref):
    x = _gat_epilogue(np_ref, dp_ref, hp_ref, pasv_ref, padv_ref, b_ref)
    x_ref[...] = x
    _h_proj(x, w_ref, as_ref, ad_ref, h_ref, asv_ref, adv_ref)


def _pool_body(np_ref, dp_ref, hp_ref, pasv_ref, padv_ref, b_ref,
               x1_ref, x2_ref, batch_ref, sums_ref, counts_ref):
    x3 = _gat_epilogue(np_ref, dp_ref, hp_ref, pasv_ref, padv_ref, b_ref)
    y = (x1_ref[...] + x2_ref[...] + x3) * (1.0 / 3.0)
    blk = batch_ref.shape[0]
    onehot = (batch_ref[...] == lax.broadcasted_iota(jnp.int32, (blk, G), 1)
              ).astype(jnp.float32)
    cdims = (((0,), (0,)), ((), ()))
    sums = lax.dot_general(onehot, y, cdims, preferred_element_type=jnp.float32)
    counts = lax.dot_general(onehot, jnp.ones((blk, 1), jnp.float32), cdims,
                             preferred_element_type=jnp.float32)

    @pl.when(pl.program_id(0) == 0)
    def _():
        sums_ref[...] = jnp.zeros_like(sums_ref)
        counts_ref[...] = jnp.zeros_like(counts_ref)

    sums_ref[...] += sums
    counts_ref[...] += counts


def _head_body(sums_ref, counts_ref, wl_ref, bl_ref, out_ref):
    pooled = sums_ref[...] / jnp.maximum(counts_ref[...], 1.0)
    logits = jnp.dot(pooled, wl_ref[...], preferred_element_type=jnp.float32)
    logits = logits + bl_ref[...]
    m = jnp.max(logits, axis=1, keepdims=True)
    z = jnp.exp(logits - m)
    out_ref[...] = z / jnp.sum(z, axis=1, keepdims=True)


def _prep1(x, W, a_s, a_d):
    return pl.pallas_call(
        _prep1_body,
        out_shape=(jax.ShapeDtypeStruct((2, NT, FH), jnp.float32),
                   jax.ShapeDtypeStruct((NT,), jnp.float32),
                   jax.ShapeDtypeStruct((NT,), jnp.float32)),
    )(x, W, a_s, a_d)


PB = 2048


def _row_specs():
    return [
        pl.BlockSpec((2, PB, FH), lambda i: (0, i, 0)),
        pl.BlockSpec((2, PB), lambda i: (0, i)),
        pl.BlockSpec((2, PB, FH), lambda i: (0, i, 0)),
        pl.BlockSpec((PB,), lambda i: (i,)),
        pl.BlockSpec((PB,), lambda i: (i,)),
        pl.BlockSpec((1, F), lambda i: (0, 0)),
    ]


def _prep_next(num_p, den_p, h_p, pasv, padv, b, W, a_s, a_d):
    return pl.pallas_call(
        _prep_next_body,
        grid=(NT // PB,),
        in_specs=_row_specs() + [
            pl.BlockSpec((F, F), lambda i: (0, 0)),
            pl.BlockSpec((F, 1), lambda i: (0, 0)),
            pl.BlockSpec((F, 1), lambda i: (0, 0)),
        ],
        out_specs=[
            pl.BlockSpec((PB, F), lambda i: (i, 0)),
            pl.BlockSpec((2, PB, FH), lambda i: (0, i, 0)),
            pl.BlockSpec((PB,), lambda i: (i,)),
            pl.BlockSpec((PB,), lambda i: (i,)),
        ],
        out_shape=(jax.ShapeDtypeStruct((NT, F), jnp.float32),
                   jax.ShapeDtypeStruct((2, NT, FH), jnp.float32),
                   jax.ShapeDtypeStruct((NT,), jnp.float32),
                   jax.ShapeDtypeStruct((NT,), jnp.float32)),
    )(num_p, den_p, h_p, pasv, padv, b, W, a_s, a_d)


def _final(num_p, den_p, h_p, pasv, padv, b, x1, x2, batch_col, Wl, bl):
    sums, counts = pl.pallas_call(
        _pool_body,
        grid=(NT // PB,),
        in_specs=_row_specs() + [
            pl.BlockSpec((PB, F), lambda i: (i, 0)),
            pl.BlockSpec((PB, F), lambda i: (i, 0)),
            pl.BlockSpec((PB, 1), lambda i: (i, 0)),
        ],
        out_specs=[
            pl.BlockSpec((G, F), lambda i: (0, 0)),
            pl.BlockSpec((G, 1), lambda i: (0, 0)),
        ],
        out_shape=(jax.ShapeDtypeStruct((G, F), jnp.float32),
                   jax.ShapeDtypeStruct((G, 1), jnp.float32)),
    )(num_p, den_p, h_p, pasv, padv, b, x1, x2, batch_col)
    return pl.pallas_call(
        _head_body,
        out_shape=jax.ShapeDtypeStruct((G, OUT), jnp.float32),
    )(sums, counts, Wl, bl)


_MESH = plsc.VectorSubcoreMesh(core_axis_name="c", subcore_axis_name="s")


def _edge_body(src_hbm, dst_hbm, h_hbm, asv_hbm, adv_hbm, z2_hbm, z1_hbm,
               num_out, den_out,
               asv_v, adv_v, idx_s, idx_d, wv, rows_g, rows_s,
               h_sp, num_sp, den_sp, sem_i, sem_g, sem_sr, sem_sw):
    cid = lax.axis_index("c")
    sid = lax.axis_index("s")
    base_n = sid * ROWS_PER_SUB

    pltpu.sync_copy(z2_hbm, num_sp.at[pl.ds(base_n, ROWS_PER_SUB)])
    pltpu.sync_copy(z1_hbm, den_sp.at[pl.ds(base_n, ROWS_PER_SUB)])
    pltpu.sync_copy(h_hbm.at[cid, pl.ds(base_n, ROWS_PER_SUB)],
                    h_sp.at[pl.ds(base_n, ROWS_PER_SUB)])
    pltpu.sync_copy(asv_hbm, asv_v)
    pltpu.sync_copy(adv_hbm, adv_v)
    plsc.subcore_barrier()

    def idx_copies(t, slot):
        base = (sid * T_CH + t) * CHUNK
        return (pltpu.make_async_copy(src_hbm.at[pl.ds(base, CHUNK)],
                                      idx_s.at[slot], sem_i.at[slot]),
                pltpu.make_async_copy(dst_hbm.at[pl.ds(base, CHUNK)],
                                      idx_d.at[slot], sem_i.at[slot]))

    def gather_copy(slot4, b2):
        return pltpu.make_async_copy(h_sp.at[idx_s.at[slot4]],
                                     rows_g.at[b2], sem_g.at[b2])

    def scatter_copies(slot4, b2):
        return (pltpu.make_async_copy(rows_s.at[b2],
                                      num_sp.at[idx_d.at[slot4]],
                                      sem_sr.at[b2]),
                pltpu.make_async_copy(wv.at[b2],
                                      den_sp.at[idx_d.at[slot4]],
                                      sem_sw.at[b2]))

    def compute_w(slot4, b2):
        for j in range(CHUNK // 16):
            si = idx_s[slot4, pl.ds(j * 16, 16)]
            di = idx_d[slot4, pl.ds(j * 16, 16)]
            e = plsc.load_gather(asv_v, [si]) + plsc.load_gather(adv_v, [di])
            e = jnp.maximum(e, 0.2 * e)
            wv[b2, pl.ds(j * 16, 16)] = jnp.exp(e)

    def scale(b2):
        def g_body(g, c2):
            wvec = wv[b2, pl.ds(g * 16, 16)]
            for el in range(16):
                i = g * 16 + el
                wb = jnp.full((16,), wvec[el], jnp.float32)
                for q in range(FH // 16):
                    rows_s[b2, i, pl.ds(q * 16, 16)] = (
                        rows_g[b2, i, pl.ds(q * 16, 16)] * wb)
            return c2

        lax.fori_loop(0, CHUNK // 16, g_body, 0, unroll=False)

    for cp in idx_copies(0, 0):
        cp.start()
        cp.wait()
    gather_copy(0, 0).start()
    for cp in idx_copies(1, 1):
        cp.start()

    def quad_body(t4, carry):
        for b in range(4):
            t = t4 * 4 + b
            b2 = b % 2
            nb2 = 1 - b2
            s_cur = b
            s_next = (b + 1) % 4
            s_pre = (b + 2) % 4

            @pl.when(t + 1 < T_CH)
            def _():
                for cp in idx_copies(t + 1, s_next):
                    cp.wait()
                gather_copy(s_next, nb2).start()

            @pl.when(t >= 2)
            def _():
                for cp in scatter_copies(s_pre, b2):
                    cp.wait()

            @pl.when(t + 2 < T_CH)
            def _():
                for cp in idx_copies(t + 2, s_pre):
                    cp.start()

            compute_w(s_cur, b2)
            gather_copy(s_cur, b2).wait()
            scale(b2)
            for cp in scatter_copies(s_cur, b2):
                cp.start(add=True)
        return carry

    lax.fori_loop(0, T_CH // 4, quad_body, 0, unroll=False)
    for cp in scatter_copies(2, 0):
        cp.wait()
    for cp in scatter_copies(3, 1):
        cp.wait()

    plsc.subcore_barrier()
    pltpu.sync_copy(num_sp.at[pl.ds(base_n, ROWS_PER_SUB)],
                    num_out.at[cid, pl.ds(base_n, ROWS_PER_SUB)])
    pltpu.sync_copy(den_sp.at[pl.ds(base_n, ROWS_PER_SUB)],
                    den_out.at[cid, pl.ds(base_n, ROWS_PER_SUB)])


_edge_agg = functools.partial(
    pl.kernel,
    mesh=_MESH,
    compiler_params=pltpu.CompilerParams(needs_layout_passes=False,
                                         use_tc_tiling_on_sc=False),
    out_type=(jax.ShapeDtypeStruct((2, NT, FH), jnp.float32),
              jax.ShapeDtypeStruct((2, NT), jnp.float32)),
    scratch_types=[
        pltpu.VMEM((NT,), jnp.float32),
        pltpu.VMEM((NT,), jnp.float32),
        pltpu.VMEM((4, CHUNK), jnp.int32),
        pltpu.VMEM((4, CHUNK), jnp.int32),
        pltpu.VMEM((2, CHUNK), jnp.float32),
        pltpu.VMEM((2, CHUNK, FH), jnp.float32),
        pltpu.VMEM((2, CHUNK, FH), jnp.float32),
        pltpu.VMEM_SHARED((NT, FH), jnp.float32),
        pltpu.VMEM_SHARED((NT, FH), jnp.float32),
        pltpu.VMEM_SHARED((NT,), jnp.float32),
        pltpu.SemaphoreType.DMA((4,)),
        pltpu.SemaphoreType.DMA((2,)),
        pltpu.SemaphoreType.DMA((2,)),
        pltpu.SemaphoreType.DMA((2,)),
    ],
)(_edge_body)


def kernel(X, L, batch, W1, a_s1, a_d1, b1, W2, a_s2, a_d2, b2,
           W3, a_s3, a_d3, b3, Wl, bl):
    ei = L[0]
    src = jnp.pad(ei[0], (0, EPAD - E), constant_values=N)
    dst = jnp.pad(ei[1], (0, EPAD - E), constant_values=N)

    x0 = jnp.pad(X[0], ((0, NT - N), (0, 0)))
    batch_col = jnp.pad(batch[0], (0, NT - N), constant_values=G).reshape(NT, 1)
    z2 = jnp.zeros((ROWS_PER_SUB, FH), jnp.float32)
    z1 = jnp.zeros((ROWS_PER_SUB,), jnp.float32)

    def layer_edges(h, asv, adv):
        return _edge_agg(src, dst, h, asv, adv, z2, z1)

    h1, asv1, adv1 = _prep1(x0, W1, a_s1.reshape(F, 1), a_d1.reshape(F, 1))
    np1, dn1 = layer_edges(h1, asv1, adv1)
    x1, h2, asv2, adv2 = _prep_next(np1, dn1, h1, asv1, adv1,
                                    b1.reshape(1, F), W2,
                                    a_s2.reshape(F, 1), a_d2.reshape(F, 1))
    np2, dn2 = layer_edges(h2, asv2, adv2)
    x2, h3, asv3, adv3 = _prep_next(np2, dn2, h2, asv2, adv2,
                                    b2.reshape(1, F), W3,
                                    a_s3.reshape(F, 1), a_d3.reshape(F, 1))
    np3, dn3 = layer_edges(h3, asv3, adv3)
    return _final(np3, dn3, h3, asv3, adv3, b3.reshape(1, F),
                  x1, x2, batch_col, Wl, bl)
